# Initial kernel scaffold; baseline (speedup 1.0000x reference)
import jax
import jax.numpy as jnp
from jax.experimental import pallas as pl


def kernel(x, embed):
    raise NotImplementedError("write your pallas kernel here")


# speedup vs baseline: 1.2025x; 1.2025x over previous
import functools

import jax
import jax.numpy as jnp
from jax import lax
from jax.experimental import pallas as pl
from jax.experimental.pallas import tpu as pltpu
from jax.experimental.pallas import tpu_sc as plsc


_N_TILE = 512


def _argmin_body(x_ref, et_ref, ind_ref):
    xt = x_ref[...]
    et = et_ref[...]
    x2 = jnp.sum(xt * xt, axis=1, keepdims=True)
    e2 = jnp.sum(et * et, axis=0, keepdims=True)
    xe = jnp.dot(xt, et, preferred_element_type=jnp.float32)
    dist = -(x2 - 2.0 * xe + e2)
    m = jnp.max(dist, axis=1, keepdims=True)
    k = dist.shape[1]
    iota = lax.broadcasted_iota(jnp.int32, dist.shape, 1)
    ind = jnp.min(jnp.where(dist == m, iota, k), axis=1, keepdims=True)
    ind_ref[...] = ind


def _argmin_indices(flat, embed_t):
    n, d = flat.shape
    k = embed_t.shape[1]
    grid = (n // _N_TILE,)
    ind = pl.pallas_call(
        _argmin_body,
        grid=grid,
        in_specs=[
            pl.BlockSpec((_N_TILE, d), lambda i: (i, 0)),
            pl.BlockSpec((d, k), lambda i: (0, 0)),
        ],
        out_specs=pl.BlockSpec((_N_TILE, 1), lambda i: (i, 0)),
        out_shape=jax.ShapeDtypeStruct((n, 1), jnp.int32),
    )(flat, embed_t)
    return ind.reshape(n)


@functools.cache
def _make_sc_gather(V, D, B):
    info = plsc.get_sparse_core_info()
    NC, NS = info.num_cores, info.num_subcores
    NW = NC * NS
    b_per_w = B // NW
    mesh = plsc.VectorSubcoreMesh(core_axis_name="c", subcore_axis_name="s")

    @functools.partial(
        pl.kernel,
        mesh=mesh,
        out_type=jax.ShapeDtypeStruct((B, D), jnp.float32),
        scratch_types=[
            pltpu.VMEM((b_per_w,), jnp.int32),
            pltpu.VMEM((b_per_w, D), jnp.float32),
            pltpu.SemaphoreType.DMA,
        ],
        compiler_params=pltpu.CompilerParams(use_tc_tiling_on_sc=False),
    )
    def gather(table_hbm, idx_hbm, out_hbm, idx_v, rows_v, sem):
        wid = lax.axis_index("s") * NC + lax.axis_index("c")
        base = wid * b_per_w
        pltpu.sync_copy(idx_hbm.at[pl.ds(base, b_per_w)], idx_v)
        pltpu.async_copy(table_hbm.at[idx_v], rows_v, sem).wait()
        pltpu.sync_copy(rows_v, out_hbm.at[pl.ds(base, b_per_w)])

    return gather


def kernel(x, embed):
    shape = x.shape
    d = shape[-1]
    flat = x.reshape(-1, d)
    n = flat.shape[0]
    k = embed.shape[0]
    ind = _argmin_indices(flat, embed.T)
    quant = _make_sc_gather(k, d, n)(embed, ind)
    return quant.reshape(shape), ind.reshape(shape[:-1])


# speedup vs baseline: 1.3302x; 1.1061x over previous
import functools

import jax
import jax.numpy as jnp
from jax import lax
from jax.experimental import pallas as pl
from jax.experimental.pallas import tpu as pltpu
from jax.experimental.pallas import tpu_sc as plsc


def _argmin_body(xt_ref, et_ref, ind3_ref):
    xb = xt_ref[0]
    et = et_ref[...]
    t = xb.shape[1]
    k = et.shape[1]
    x2 = jnp.sum(xb * xb, axis=0, keepdims=True)
    e2 = jnp.sum(et * et, axis=0, keepdims=True)
    e2c = jnp.transpose(e2)
    et2 = et + et
    xe2 = lax.dot_general(
        et2, xb, (((0,), (0,)), ((), ())),
        preferred_element_type=jnp.float32)
    pre = (x2 - xe2) + e2c
    m = jnp.min(pre, axis=0, keepdims=True)
    iota = lax.broadcasted_iota(jnp.int32, (k, t), 0)
    ind = jnp.min(jnp.where(pre == m, iota, k), axis=0, keepdims=True)
    ind3_ref[...] = ind.reshape(1, 1, t)


def _argmin_indices(x_t, embed_t):
    b, d, t = x_t.shape
    k = embed_t.shape[1]
    ind3 = pl.pallas_call(
        _argmin_body,
        grid=(b,),
        in_specs=[
            pl.BlockSpec((1, d, t), lambda i: (i, 0, 0)),
            pl.BlockSpec((d, k), lambda i: (0, 0)),
        ],
        out_specs=pl.BlockSpec((1, 1, t), lambda i: (i, 0, 0)),
        out_shape=jax.ShapeDtypeStruct((b, 1, t), jnp.int32),
    )(x_t, embed_t)
    return ind3


@functools.cache
def _make_sc_gather(V, D, B):
    info = plsc.get_sparse_core_info()
    NC, NS = info.num_cores, info.num_subcores
    NW = NC * NS
    b_per_w = B // NW
    mesh = plsc.VectorSubcoreMesh(core_axis_name="c", subcore_axis_name="s")

    @functools.partial(
        pl.kernel,
        mesh=mesh,
        out_type=jax.ShapeDtypeStruct((B, D), jnp.float32),
        scratch_types=[
            pltpu.VMEM((b_per_w,), jnp.int32),
            pltpu.VMEM((b_per_w, D), jnp.float32),
            pltpu.SemaphoreType.DMA,
        ],
        compiler_params=pltpu.CompilerParams(use_tc_tiling_on_sc=False),
    )
    def gather(table_hbm, idx_hbm, out_hbm, idx_v, rows_v, sem):
        wid = lax.axis_index("s") * NC + lax.axis_index("c")
        base = wid * b_per_w
        pltpu.sync_copy(idx_hbm.at[pl.ds(base, b_per_w)], idx_v)
        pltpu.async_copy(table_hbm.at[idx_v], rows_v, sem).wait()
        pltpu.sync_copy(rows_v, out_hbm.at[pl.ds(base, b_per_w)])

    return gather


def kernel(x, embed):
    b, t, d = x.shape
    k = embed.shape[0]
    x_t = x.transpose(0, 2, 1)
    embed_t = embed.T
    ind3 = _argmin_indices(x_t, embed_t)
    ind1 = ind3.reshape(b * t)
    quant = _make_sc_gather(k, d, b * t)(embed, ind1)
    return quant.reshape(b, t, d), ind3.reshape(b, t)


# speedup vs baseline: 1.6335x; 1.2281x over previous
import functools

import jax
import jax.numpy as jnp
from jax import lax
from jax.experimental import pallas as pl
from jax.experimental.pallas import tpu as pltpu
from jax.experimental.pallas import tpu_sc as plsc


_B_BLK = 4


def _argmin_body(xt_ref, et_ref, ind3_ref, e2c_ref, et2_ref):
    t = xt_ref.shape[2]
    k = et_ref.shape[1]

    @pl.when(pl.program_id(0) == 0)
    def _():
        et = et_ref[...]
        e2 = jnp.sum(et * et, axis=0, keepdims=True)
        e2c_ref[...] = jnp.transpose(e2)
        et2_ref[...] = et + et

    e2c = e2c_ref[...]
    et2 = et2_ref[...]
    iota_f = lax.broadcasted_iota(jnp.int32, (k, t), 0).astype(jnp.float32)
    for b in range(_B_BLK):
        xb = xt_ref[b]
        x2 = jnp.sum(xb * xb, axis=0, keepdims=True)
        xe2 = lax.dot_general(
            et2, xb, (((0,), (0,)), ((), ())),
            preferred_element_type=jnp.float32)
        pre = (x2 - xe2) + e2c
        m = jnp.min(pre, axis=0, keepdims=True)
        indf = jnp.min(jnp.where(pre == m, iota_f, float(k)),
                       axis=0, keepdims=True)
        ind3_ref[b] = indf.astype(jnp.int32)


def _argmin_indices(x_t, embed_t):
    b, d, t = x_t.shape
    k = embed_t.shape[1]
    ind3 = pl.pallas_call(
        _argmin_body,
        grid=(b // _B_BLK,),
        in_specs=[
            pl.BlockSpec((_B_BLK, d, t), lambda i: (i, 0, 0)),
            pl.BlockSpec((d, k), lambda i: (0, 0)),
        ],
        out_specs=pl.BlockSpec((_B_BLK, 1, t), lambda i: (i, 0, 0)),
        out_shape=jax.ShapeDtypeStruct((b, 1, t), jnp.int32),
        scratch_shapes=[
            pltpu.VMEM((k, 1), jnp.float32),
            pltpu.VMEM((d, k), jnp.float32),
        ],
    )(x_t, embed_t)
    return ind3


@functools.cache
def _make_sc_gather(V, D, B):
    info = plsc.get_sparse_core_info()
    NC, NS = info.num_cores, info.num_subcores
    NW = NC * NS
    b_per_w = B // NW
    mesh = plsc.VectorSubcoreMesh(core_axis_name="c", subcore_axis_name="s")

    @functools.partial(
        pl.kernel,
        mesh=mesh,
        out_type=jax.ShapeDtypeStruct((B, D), jnp.float32),
        scratch_types=[
            pltpu.VMEM((b_per_w,), jnp.int32),
            pltpu.VMEM((b_per_w, D), jnp.float32),
            pltpu.SemaphoreType.DMA,
        ],
        compiler_params=pltpu.CompilerParams(use_tc_tiling_on_sc=False),
    )
    def gather(table_hbm, idx_hbm, out_hbm, idx_v, rows_v, sem):
        wid = lax.axis_index("s") * NC + lax.axis_index("c")
        base = wid * b_per_w
        pltpu.sync_copy(idx_hbm.at[pl.ds(base, b_per_w)], idx_v)
        pltpu.async_copy(table_hbm.at[idx_v], rows_v, sem).wait()
        pltpu.sync_copy(rows_v, out_hbm.at[pl.ds(base, b_per_w)])

    return gather


def kernel(x, embed):
    b, t, d = x.shape
    k = embed.shape[0]
    x_t = x.transpose(0, 2, 1)
    embed_t = embed.T
    ind3 = _argmin_indices(x_t, embed_t)
    ind1 = ind3.reshape(b * t)
    quant = _make_sc_gather(k, d, b * t)(embed, ind1)
    return quant.reshape(b, t, d), ind3.reshape(b, t)


# speedup vs baseline: 1.8430x; 1.1282x over previous
import functools

import jax
import jax.numpy as jnp
from jax import lax
from jax.experimental import pallas as pl
from jax.experimental.pallas import tpu as pltpu
from jax.experimental.pallas import tpu_sc as plsc


_B_BLK = 4


def _argmin_body(xt_ref, et_ref, ind3_ref, e2c_ref, et2_ref):
    t = xt_ref.shape[2]
    k = et_ref.shape[1]

    @pl.when(pl.program_id(0) == 0)
    def _():
        et = et_ref[...]
        e2 = jnp.sum(et * et, axis=0, keepdims=True)
        e2c_ref[...] = jnp.transpose(e2)
        et2_ref[...] = et + et

    e2c = e2c_ref[...]
    et2 = et2_ref[...]
    iota_f = lax.broadcasted_iota(jnp.int32, (k, t), 0).astype(jnp.float32)
    for b in range(_B_BLK):
        xb = xt_ref[b]
        x2 = jnp.sum(xb * xb, axis=0, keepdims=True)
        xe2 = lax.dot_general(
            et2, xb, (((0,), (0,)), ((), ())),
            preferred_element_type=jnp.float32)
        pre = (x2 - xe2) + e2c
        m = jnp.min(pre, axis=0, keepdims=True)
        indf = jnp.min(jnp.where(pre == m, iota_f, float(k)),
                       axis=0, keepdims=True)
        ind3_ref[b] = indf.astype(jnp.int32)


def _argmin_indices(x_t, embed_t):
    b, d, t = x_t.shape
    k = embed_t.shape[1]
    ind3 = pl.pallas_call(
        _argmin_body,
        grid=(b // _B_BLK,),
        in_specs=[
            pl.BlockSpec((_B_BLK, d, t), lambda i: (i, 0, 0)),
            pl.BlockSpec((d, k), lambda i: (0, 0)),
        ],
        out_specs=pl.BlockSpec((_B_BLK, 1, t), lambda i: (i, 0, 0)),
        out_shape=jax.ShapeDtypeStruct((b, 1, t), jnp.int32),
        scratch_shapes=[
            pltpu.VMEM((k, 1), jnp.float32),
            pltpu.VMEM((d, k), jnp.float32),
        ],
    )(x_t, embed_t)
    return ind3


@functools.cache
def _make_sc_gather(K, D, B, T):
    info = plsc.get_sparse_core_info()
    NC, NS, L = info.num_cores, info.num_subcores, info.num_lanes
    NW = NC * NS
    assert B == NW and T % L == 0
    mesh = plsc.VectorSubcoreMesh(core_axis_name="c", subcore_axis_name="s")

    @functools.partial(
        pl.kernel,
        mesh=mesh,
        out_type=jax.ShapeDtypeStruct((B, D, T), jnp.float32),
        scratch_types=[
            pltpu.VMEM((T,), jnp.int32),
            pltpu.VMEM((K * D,), jnp.float32),
            pltpu.VMEM((D, T), jnp.float32),
        ],
        compiler_params=pltpu.CompilerParams(needs_layout_passes=False),
    )
    def gather(et_flat_hbm, idx_hbm, out_hbm, idx_v, tab_v, out_v):
        wid = lax.axis_index("s") * NC + lax.axis_index("c")
        base = wid * T
        pltpu.sync_copy(idx_hbm.at[pl.ds(base, T)], idx_v)
        pltpu.sync_copy(et_flat_hbm, tab_v)

        def body(g, _):
            fi = idx_v[pl.ds(g * L, L)]
            for d in range(D):
                out_v[d, pl.ds(g * L, L)] = plsc.load_gather(tab_v, [fi])
                fi = fi + K
            return _

        lax.fori_loop(0, T // L, body, None, unroll=False)
        pltpu.sync_copy(out_v, out_hbm.at[wid])

    return gather


def kernel(x, embed):
    b, t, d = x.shape
    k = embed.shape[0]
    x_t = x.transpose(0, 2, 1)
    embed_t = embed.T
    ind3 = _argmin_indices(x_t, embed_t)
    ind1 = ind3.reshape(b * t)
    quant_t = _make_sc_gather(k, d, b, t)(embed_t.reshape(k * d), ind1)
    return quant_t.transpose(0, 2, 1), ind3.reshape(b, t)
